# speedup vs baseline: 1.1531x; 1.1531x over previous
import jax
import jax.numpy as jnp
from jax.experimental import pallas as pl

_P = 1024


def _vq_body(z_ref, zn_ref, emb_ref, zq_ref, idx_ref):
    zb = z_ref[...]
    e = emb_ref[...]
    en = jnp.sum(e * e, axis=1)[None, :]
    s = jax.lax.dot_general(
        zb, e, (((1,), (1,)), ((), ())),
        preferred_element_type=jnp.float32)
    d = (zn_ref[...] + en) - 2.0 * s
    m = jnp.min(d, axis=1, keepdims=True)
    ji = jax.lax.broadcasted_iota(jnp.int32, d.shape, 1)
    idx = jnp.min(jnp.where(d == m, ji, d.shape[1]), axis=1)
    idx_ref[0, 0, :] = idx
    onehot_t = (jax.lax.broadcasted_iota(jnp.int32, (e.shape[0], _P), 0)
                == idx[None, :]).astype(jnp.float32)
    zq_ref[0, :, :] = jax.lax.dot_general(
        e, onehot_t, (((0,), (0,)), ((), ())),
        preferred_element_type=jnp.float32,
        precision=jax.lax.Precision.HIGHEST)


def kernel(z, emb_weight):
    B, C, H, W = z.shape
    N = B * H * W
    J = emb_weight.shape[0]
    z_flat = jnp.transpose(z, (0, 2, 3, 1)).reshape(N, C)
    znorm = jnp.sum(z_flat ** 2, axis=1, keepdims=True)

    grid = (N // _P,)
    zq_t, idx = pl.pallas_call(
        _vq_body,
        grid=grid,
        in_specs=[
            pl.BlockSpec((_P, C), lambda b: (b, 0)),
            pl.BlockSpec((_P, 1), lambda b: (b, 0)),
            pl.BlockSpec((J, C), lambda b: (0, 0)),
        ],
        out_specs=[
            pl.BlockSpec((1, C, _P), lambda b: (b, 0, 0)),
            pl.BlockSpec((1, 1, _P), lambda b: (b, 0, 0)),
        ],
        out_shape=[
            jax.ShapeDtypeStruct((N // _P, C, _P), jnp.float32),
            jax.ShapeDtypeStruct((N // _P, 1, _P), jnp.int32),
        ],
    )(z_flat, znorm, emb_weight)

    hw = H * W
    z_q = zq_t.reshape(B, hw // _P, C, _P)
    z_q = jnp.transpose(z_q, (0, 2, 1, 3)).reshape(B, C, H, W)
    min_idx = idx.reshape(B, H, W)
    return (z_q, min_idx)
